# baseline (speedup 1.0000x reference)
import functools

import jax
import jax.numpy as jnp
import numpy as np
from jax import lax
from jax.experimental import pallas as pl
from jax.experimental.pallas import tpu as pltpu
from jax.experimental.pallas import tpu_sc as plsc

EPS = 1e-5
NPOINT = 512
NSAMPLE = 32


def _index_points(points, idx):
    return jax.vmap(lambda p, i: p[i])(points, idx)


def _fps_kernel(x_ref, y_ref, z_ref, far0_ref, idx_ref, cx_ref, cy_ref, cz_ref):
    B, N = x_ref.shape
    iota_l = jax.lax.broadcasted_iota(jnp.int32, (B, N), 1)
    iota_p = jax.lax.broadcasted_iota(jnp.int32, (B, NPOINT), 1)
    x = x_ref[...]
    y = y_ref[...]
    z = z_ref[...]

    def body(i, st):
        dist, far, idx_acc, cx_acc, cy_acc, cz_acc = st
        mask = iota_l == far
        cx = jnp.sum(jnp.where(mask, x, 0.0), axis=1, keepdims=True)
        cy = jnp.sum(jnp.where(mask, y, 0.0), axis=1, keepdims=True)
        cz = jnp.sum(jnp.where(mask, z, 0.0), axis=1, keepdims=True)
        sel = iota_p == i
        idx_acc = jnp.where(sel, far, idx_acc)
        cx_acc = jnp.where(sel, cx, cx_acc)
        cy_acc = jnp.where(sel, cy, cy_acc)
        cz_acc = jnp.where(sel, cz, cz_acc)
        dx = x - cx
        dy = y - cy
        dz = z - cz
        d = (dx * dx + dy * dy) + dz * dz
        dist = jnp.minimum(dist, d)
        m = jnp.max(dist, axis=1, keepdims=True)
        far = jnp.min(jnp.where(dist == m, iota_l, N), axis=1, keepdims=True)
        return (dist, far, idx_acc, cx_acc, cy_acc, cz_acc)

    dist0 = jnp.full((B, N), 1e10, dtype=jnp.float32)
    zp = jnp.zeros((B, NPOINT), dtype=jnp.float32)
    zi = jnp.zeros((B, NPOINT), dtype=jnp.int32)
    _, _, idx_acc, cx_acc, cy_acc, cz_acc = jax.lax.fori_loop(
        0, NPOINT, body, (dist0, far0_ref[...], zi, zp, zp, zp))
    idx_ref[...] = idx_acc
    cx_ref[...] = cx_acc
    cy_ref[...] = cy_acc
    cz_ref[...] = cz_acc


def _fps_pallas(xyz_p):
    B, N, _ = xyz_p.shape
    far0 = jax.random.randint(jax.random.key(42), (B,), 0, N).astype(jnp.int32)[:, None]
    x = xyz_p[:, :, 0]
    y = xyz_p[:, :, 1]
    z = xyz_p[:, :, 2]
    idx, cx, cy, cz = pl.pallas_call(
        _fps_kernel,
        out_shape=(
            jax.ShapeDtypeStruct((B, NPOINT), jnp.int32),
            jax.ShapeDtypeStruct((B, NPOINT), jnp.float32),
            jax.ShapeDtypeStruct((B, NPOINT), jnp.float32),
            jax.ShapeDtypeStruct((B, NPOINT), jnp.float32),
        ),
    )(x, y, z, far0)
    new_xyz = jnp.stack([cx, cy, cz], axis=2)
    return idx, new_xyz


def _knn_kernel(nx8_ref, xyzT8_ref, idx_ref):
    M, N = 512, 4096
    nx8 = nx8_ref[0]
    xyzT8 = xyzT8_ref[0]
    mm = jax.lax.dot_general(nx8, xyzT8, (((1,), (0,)), ((), ())),
                             preferred_element_type=jnp.float32)
    sqr = -2.0 * mm
    sqr = sqr + jnp.sum(nx8 * nx8, axis=1, keepdims=True)
    sqr = sqr + jnp.sum(xyzT8 * xyzT8, axis=0, keepdims=True)
    iota_l = jax.lax.broadcasted_iota(jnp.int32, (M, N), 1)
    cols = []
    for _ in range(NSAMPLE):
        m = jnp.min(sqr, axis=1, keepdims=True)
        sel = jnp.min(jnp.where(sqr == m, iota_l, N), axis=1, keepdims=True)
        cols.append(sel)
        sqr = jnp.where(iota_l == sel, jnp.inf, sqr)
    idx_ref[0] = jnp.concatenate(cols, axis=1)


def _knn_pallas(xyz_p, new_xyz):
    B, N, _ = xyz_p.shape
    nx8 = jnp.concatenate([new_xyz, jnp.zeros((B, NPOINT, 5), jnp.float32)], axis=2)
    xyzT8 = jnp.concatenate([xyz_p.transpose(0, 2, 1), jnp.zeros((B, 5, N), jnp.float32)], axis=1)
    idx = pl.pallas_call(
        _knn_kernel,
        grid=(B,),
        in_specs=[
            pl.BlockSpec((1, NPOINT, 8), lambda b: (b, 0, 0)),
            pl.BlockSpec((1, 8, N), lambda b: (b, 0, 0)),
        ],
        out_specs=pl.BlockSpec((1, NPOINT, NSAMPLE), lambda b: (b, 0, 0)),
        out_shape=jax.ShapeDtypeStruct((B, NPOINT, NSAMPLE), jnp.int32),
    )(nx8, xyzT8)
    return idx


def _sc_gather_rows(table, gidx, ncols):
    NR = gidx.shape[0]
    NW = 32
    rows_per_w = NR // NW
    CHUNK = 128
    n_chunks = rows_per_w // CHUNK
    mesh = plsc.VectorSubcoreMesh(core_axis_name="c", subcore_axis_name="s")

    @functools.partial(
        pl.kernel,
        mesh=mesh,
        out_type=jax.ShapeDtypeStruct((NR, ncols), jnp.float32),
        scratch_types=[
            pltpu.VMEM((CHUNK,), jnp.int32),
            pltpu.VMEM((CHUNK, ncols), jnp.float32),
            pltpu.SemaphoreType.DMA,
        ],
        compiler_params=pltpu.CompilerParams(use_tc_tiling_on_sc=False),
    )
    def k(table_hbm, gidx_hbm, out_hbm, idx_v, rows_v, sem):
        wid = lax.axis_index("s") * 2 + lax.axis_index("c")
        base = wid * rows_per_w

        def chunk_body(ci, _):
            cb = base + ci * CHUNK
            pltpu.sync_copy(gidx_hbm.at[pl.ds(cb, CHUNK)], idx_v)
            pltpu.async_copy(table_hbm.at[idx_v], rows_v, sem).wait()
            pltpu.sync_copy(rows_v, out_hbm.at[pl.ds(cb, CHUNK)])
            return 0

        lax.fori_loop(0, n_chunks, chunk_body, 0)

    return k(table, gidx)


NTOT = float(8 * NPOINT * NSAMPLE)
TILE = 512


def _tile_stats(x):
    s = jnp.sum(x, axis=0, keepdims=True)
    sq = jnp.sum(x * x, axis=0, keepdims=True)
    return jnp.concatenate([s, sq], axis=0)


def _acc_stats(ref, x):
    @pl.when(pl.program_id(0) == 0)
    def _():
        ref[...] = jnp.zeros_like(ref)
    ref[...] += _tile_stats(x)


def _bn_consts(stats, g, b):
    mean = stats[0:1] / NTOT
    var = stats[1:2] / NTOT - mean * mean
    a = g / jnp.sqrt(var + EPS)
    c = b - mean * a
    return a, c


def _k0_body(g_ref, nxe_ref, wx_ref, wf_ref, b0_ref, wwx_ref, bw0_ref,
             out_ref, wout_ref, st_ref, wst_ref):
    gt = g_ref[...]
    xn16 = gt[:, 0:16] - nxe_ref[...]
    out0 = (jax.lax.dot_general(xn16, wx_ref[...], (((1,), (0,)), ((), ())),
                                preferred_element_type=jnp.float32)
            + jax.lax.dot_general(gt, wf_ref[...], (((1,), (0,)), ((), ())),
                                  preferred_element_type=jnp.float32)
            + b0_ref[...])
    wout0 = jax.lax.dot_general(xn16, wwx_ref[...], (((1,), (0,)), ((), ())),
                                preferred_element_type=jnp.float32) + bw0_ref[...]
    out_ref[...] = out0
    wout_ref[...] = wout0
    _acc_stats(st_ref, out0)
    _acc_stats(wst_ref, wout0)


def _klayer_body(x_ref, wx_in_ref, st_in_ref, wst_in_ref, g_ref, bb_ref,
                 wg_ref, wbb_ref, w_ref, b_ref, ww_ref, wb_ref,
                 out_ref, wout_ref, st_ref, wst_ref):
    a, c = _bn_consts(st_in_ref[...], g_ref[...], bb_ref[...])
    x = jnp.maximum(x_ref[...] * a + c, 0.0)
    wa, wc = _bn_consts(wst_in_ref[...], wg_ref[...], wbb_ref[...])
    wx = jnp.maximum(wx_in_ref[...] * wa + wc, 0.0)
    out = jax.lax.dot_general(x, w_ref[...], (((1,), (0,)), ((), ())),
                              preferred_element_type=jnp.float32) + b_ref[...]
    wout = jax.lax.dot_general(wx, ww_ref[...], (((1,), (0,)), ((), ())),
                               preferred_element_type=jnp.float32) + wb_ref[...]
    out_ref[...] = out
    wout_ref[...] = wout
    _acc_stats(st_ref, out)
    _acc_stats(wst_ref, wout)


_NB = 64


def _k3_body(x_ref, wx_ref, st_ref, wst_ref, g_ref, bb_ref, wg_ref, wbb_ref, gt_ref):
    a, c = _bn_consts(st_ref[...], g_ref[...], bb_ref[...])
    x = jnp.maximum(x_ref[...] * a + c, 0.0)
    wa, wc = _bn_consts(wst_ref[...], wg_ref[...], wbb_ref[...])
    wx = jnp.maximum(wx_ref[...] * wa + wc, 0.0)
    for i in range(_NB):
        xi = x[i * NSAMPLE:(i + 1) * NSAMPLE]
        wi = wx[i * NSAMPLE:(i + 1) * NSAMPLE]
        gt_ref[:, i, :] = jax.lax.dot_general(
            wi, xi, (((0,), (0,)), ((), ())), preferred_element_type=jnp.float32)


def _k4_body(gt_ref, w3_ref, lb_ref, lin_ref, st_ref, acc_ref):
    b = pl.program_id(0)
    j = pl.program_id(1)

    @pl.when(j == 0)
    def _():
        acc_ref[...] = jnp.broadcast_to(lb_ref[...], acc_ref.shape)

    acc_ref[...] += jax.lax.dot_general(
        gt_ref[0], w3_ref[0], (((1,), (0,)), ((), ())),
        preferred_element_type=jnp.float32)

    @pl.when(j == 15)
    def _():
        a = acc_ref[...]
        lin_ref[...] = a

        @pl.when(b == 0)
        def _():
            st_ref[...] = jnp.zeros_like(st_ref)

        st_ref[...] += _tile_stats(a)


def _k5_body(x_ref, st_ref, g_ref, bb_ref, o_ref):
    a, c = _bn_consts(st_ref[...] * (NTOT / (8.0 * NPOINT)), g_ref[...], bb_ref[...])
    o_ref[...] = jnp.maximum(x_ref[...] * a + c, 0.0)


def _mlp_pallas(G, new_xyz, params):
    R = G.shape[0]
    nsteps = R // TILE

    def pad_rows(m, rows, at, total):
        z0 = jnp.zeros((at, m.shape[1]), jnp.float32)
        z1 = jnp.zeros((total - at - rows, m.shape[1]), jnp.float32)
        return jnp.concatenate([z0, m, z1], axis=0)

    def pad_cols(v, total):
        return jnp.concatenate([v, jnp.zeros((total - v.shape[0],), jnp.float32)])

    w0t = params['conv0_w'].T
    wx16 = pad_rows(w0t[0:3], 3, 0, 16)
    wf144 = pad_rows(w0t[3:131], 128, 3, 144)
    b0 = params['conv0_b'][None, :]
    ww0 = jnp.pad(params['wconv0_w'].T, ((0, 13), (0, 8)))
    bw0 = pad_cols(params['wconv0_b'], 16)[None, :]
    nxe = jnp.repeat(
        jnp.concatenate([new_xyz, jnp.zeros((8, NPOINT, 13), jnp.float32)],
                        axis=2).reshape(8 * NPOINT, 16), NSAMPLE, axis=0)

    row_spec = lambda w: pl.BlockSpec((TILE, w), lambda i: (i, 0))
    full_spec = lambda a: pl.BlockSpec(a.shape, lambda i: tuple(0 for _ in a.shape))
    stat_spec = lambda w: pl.BlockSpec((2, w), lambda i: (0, 0))

    out0, wout0, st0, wst0 = pl.pallas_call(
        _k0_body,
        grid=(nsteps,),
        in_specs=[row_spec(144), row_spec(16)] + [full_spec(a) for a in (wx16, wf144, b0, ww0, bw0)],
        out_specs=(row_spec(128), row_spec(16), stat_spec(128), stat_spec(16)),
        out_shape=(jax.ShapeDtypeStruct((R, 128), jnp.float32),
                   jax.ShapeDtypeStruct((R, 16), jnp.float32),
                   jax.ShapeDtypeStruct((2, 128), jnp.float32),
                   jax.ShapeDtypeStruct((2, 16), jnp.float32)),
    )(G, nxe, wx16, wf144, b0, ww0, bw0)

    def layer(i, x, wx, st, wst, oc):
        ic = x.shape[1]
        g = params['bn%d_g' % (i - 1)][None, :]
        bb = params['bn%d_b' % (i - 1)][None, :]
        wg = pad_cols(params['wbn%d_g' % (i - 1)], 16)[None, :]
        wbb = pad_cols(params['wbn%d_b' % (i - 1)], 16)[None, :]
        w = params['conv%d_w' % i].T
        b = params['conv%d_b' % i][None, :]
        wwt = params['wconv%d_w' % i].T
        ww = jnp.pad(wwt, ((0, 16 - wwt.shape[0]), (0, 16 - wwt.shape[1])))
        wb = pad_cols(params['wconv%d_b' % i], 16)[None, :]
        return pl.pallas_call(
            _klayer_body,
            grid=(nsteps,),
            in_specs=[row_spec(ic), row_spec(16), stat_spec(ic), stat_spec(16)]
                     + [full_spec(a) for a in (g, bb, wg, wbb, w, b, ww, wb)],
            out_specs=(row_spec(oc), row_spec(16), stat_spec(oc), stat_spec(16)),
            out_shape=(jax.ShapeDtypeStruct((R, oc), jnp.float32),
                       jax.ShapeDtypeStruct((R, 16), jnp.float32),
                       jax.ShapeDtypeStruct((2, oc), jnp.float32),
                       jax.ShapeDtypeStruct((2, 16), jnp.float32)),
        )(x, wx, st, wst, g, bb, wg, wbb, w, b, ww, wb)

    out1, wout1, st1, wst1 = layer(1, out0, wout0, st0, wst0, 128)
    out2, wout2, st2, wst2 = layer(2, out1, wout1, st1, wst1, 256)

    g2 = params['bn2_g'][None, :]
    bb2 = params['bn2_b'][None, :]
    wg2 = pad_cols(params['wbn2_g'], 16)[None, :]
    wbb2 = pad_cols(params['wbn2_b'], 16)[None, :]
    n_total = 8 * NPOINT
    gt = pl.pallas_call(
        _k3_body,
        grid=(n_total // _NB,),
        in_specs=[pl.BlockSpec((_NB * NSAMPLE, 256), lambda i: (i, 0)),
                  pl.BlockSpec((_NB * NSAMPLE, 16), lambda i: (i, 0)),
                  stat_spec(256), stat_spec(16),
                  full_spec(g2), full_spec(bb2), full_spec(wg2), full_spec(wbb2)],
        out_specs=pl.BlockSpec((16, _NB, 256), lambda i: (0, i, 0)),
        out_shape=jax.ShapeDtypeStruct((16, n_total, 256), jnp.float32),
    )(out2, wout2, st2, wst2, g2, bb2, wg2, wbb2)

    w3 = params['lin_w'].reshape(256, 256, 16).transpose(2, 1, 0)
    lb = params['lin_b'][None, :]
    lin, stl = pl.pallas_call(
        _k4_body,
        grid=(8, 16),
        in_specs=[pl.BlockSpec((1, NPOINT, 256), lambda b, j: (j, b, 0)),
                  pl.BlockSpec((1, 256, 256), lambda b, j: (j, 0, 0)),
                  pl.BlockSpec((1, 256), lambda b, j: (0, 0))],
        out_specs=(pl.BlockSpec((NPOINT, 256), lambda b, j: (b, 0)),
                   pl.BlockSpec((2, 256), lambda b, j: (0, 0))),
        out_shape=(jax.ShapeDtypeStruct((8 * NPOINT, 256), jnp.float32),
                   jax.ShapeDtypeStruct((2, 256), jnp.float32)),
        scratch_shapes=[pltpu.VMEM((NPOINT, 256), jnp.float32)],
    )(gt, w3, lb)

    gl = params['bnl_g'][None, :]
    bl = params['bnl_b'][None, :]
    out = pl.pallas_call(
        _k5_body,
        grid=(8,),
        in_specs=[pl.BlockSpec((NPOINT, 256), lambda b: (b, 0)),
                  pl.BlockSpec((2, 256), lambda b: (0, 0)),
                  full_spec(gl), full_spec(bl)],
        out_specs=pl.BlockSpec((NPOINT, 256), lambda b: (b, 0)),
        out_shape=jax.ShapeDtypeStruct((8 * NPOINT, 256), jnp.float32),
    )(lin, stl, gl, bl)
    return out.reshape(8, NPOINT, 256)


def kernel(xyz, points, params):
    B = xyz.shape[0]
    xyz_p = xyz.transpose(0, 2, 1)
    pts_p = points.transpose(0, 2, 1)
    fps_idx, new_xyz = _fps_pallas(xyz_p)
    idx = _knn_pallas(xyz_p, new_xyz)
    N = xyz_p.shape[1]
    table = jnp.concatenate(
        [xyz_p, pts_p, jnp.zeros((B, N, 13), jnp.float32)], axis=2).reshape(B * N, 144)
    gidx = (idx + (jnp.arange(B, dtype=jnp.int32) * N)[:, None, None]).reshape(-1)
    G = _sc_gather_rows(table, gidx, 144)
    out = _mlp_pallas(G, new_xyz, params)
    return (new_xyz.transpose(0, 2, 1), out.transpose(0, 2, 1))


# speedup vs baseline: 1.0308x; 1.0308x over previous
import functools

import jax
import jax.numpy as jnp
import numpy as np
from jax import lax
from jax.experimental import pallas as pl
from jax.experimental.pallas import tpu as pltpu
from jax.experimental.pallas import tpu_sc as plsc

EPS = 1e-5
NPOINT = 512
NSAMPLE = 32


def _index_points(points, idx):
    return jax.vmap(lambda p, i: p[i])(points, idx)


def _fps_kernel(x_ref, y_ref, z_ref, far0_ref, idx_ref, cx_ref, cy_ref, cz_ref):
    B, N = x_ref.shape
    iota_l = jax.lax.broadcasted_iota(jnp.int32, (B, N), 1)
    iota_p = jax.lax.broadcasted_iota(jnp.int32, (B, NPOINT), 1)
    x = x_ref[...]
    y = y_ref[...]
    z = z_ref[...]

    def body(i, st):
        dist, far, idx_acc, cx_acc, cy_acc, cz_acc = st
        mask = iota_l == far
        cx = jnp.sum(jnp.where(mask, x, 0.0), axis=1, keepdims=True)
        cy = jnp.sum(jnp.where(mask, y, 0.0), axis=1, keepdims=True)
        cz = jnp.sum(jnp.where(mask, z, 0.0), axis=1, keepdims=True)
        sel = iota_p == i
        idx_acc = jnp.where(sel, far, idx_acc)
        cx_acc = jnp.where(sel, cx, cx_acc)
        cy_acc = jnp.where(sel, cy, cy_acc)
        cz_acc = jnp.where(sel, cz, cz_acc)
        dx = x - cx
        dy = y - cy
        dz = z - cz
        d = (dx * dx + dy * dy) + dz * dz
        dist = jnp.minimum(dist, d)
        m = jnp.max(dist, axis=1, keepdims=True)
        far = jnp.min(jnp.where(dist == m, iota_l, N), axis=1, keepdims=True)
        return (dist, far, idx_acc, cx_acc, cy_acc, cz_acc)

    dist0 = jnp.full((B, N), 1e10, dtype=jnp.float32)
    zp = jnp.zeros((B, NPOINT), dtype=jnp.float32)
    zi = jnp.zeros((B, NPOINT), dtype=jnp.int32)
    _, _, idx_acc, cx_acc, cy_acc, cz_acc = jax.lax.fori_loop(
        0, NPOINT, body, (dist0, far0_ref[...], zi, zp, zp, zp))
    idx_ref[...] = idx_acc
    cx_ref[...] = cx_acc
    cy_ref[...] = cy_acc
    cz_ref[...] = cz_acc


def _fps_pallas(xyz_p):
    B, N, _ = xyz_p.shape
    far0 = jax.random.randint(jax.random.key(42), (B,), 0, N).astype(jnp.int32)[:, None]
    x = xyz_p[:, :, 0]
    y = xyz_p[:, :, 1]
    z = xyz_p[:, :, 2]
    idx, cx, cy, cz = pl.pallas_call(
        _fps_kernel,
        out_shape=(
            jax.ShapeDtypeStruct((B, NPOINT), jnp.int32),
            jax.ShapeDtypeStruct((B, NPOINT), jnp.float32),
            jax.ShapeDtypeStruct((B, NPOINT), jnp.float32),
            jax.ShapeDtypeStruct((B, NPOINT), jnp.float32),
        ),
    )(x, y, z, far0)
    new_xyz = jnp.stack([cx, cy, cz], axis=2)
    return idx, new_xyz


def _knn_kernel(nx8_ref, xyzT8_ref, idx_ref):
    M, N = 512, 4096
    nx8 = nx8_ref[0]
    xyzT8 = xyzT8_ref[0]
    mm = jax.lax.dot_general(nx8, xyzT8, (((1,), (0,)), ((), ())),
                             preferred_element_type=jnp.float32)
    sqr = -2.0 * mm
    sqr = sqr + jnp.sum(nx8 * nx8, axis=1, keepdims=True)
    sqr = sqr + jnp.sum(xyzT8 * xyzT8, axis=0, keepdims=True)
    iota_l = jax.lax.broadcasted_iota(jnp.int32, (M, N), 1)
    cols = []
    for _ in range(NSAMPLE):
        sel = jnp.argmin(sqr, axis=1).astype(jnp.int32)[:, None]
        cols.append(sel)
        sqr = jnp.where(iota_l == sel, jnp.inf, sqr)
    idx_ref[0] = jnp.concatenate(cols, axis=1)


def _knn_pallas(xyz_p, new_xyz):
    B, N, _ = xyz_p.shape
    nx8 = jnp.concatenate([new_xyz, jnp.zeros((B, NPOINT, 5), jnp.float32)], axis=2)
    xyzT8 = jnp.concatenate([xyz_p.transpose(0, 2, 1), jnp.zeros((B, 5, N), jnp.float32)], axis=1)
    idx = pl.pallas_call(
        _knn_kernel,
        grid=(B,),
        in_specs=[
            pl.BlockSpec((1, NPOINT, 8), lambda b: (b, 0, 0)),
            pl.BlockSpec((1, 8, N), lambda b: (b, 0, 0)),
        ],
        out_specs=pl.BlockSpec((1, NPOINT, NSAMPLE), lambda b: (b, 0, 0)),
        out_shape=jax.ShapeDtypeStruct((B, NPOINT, NSAMPLE), jnp.int32),
    )(nx8, xyzT8)
    return idx


def _sc_gather_rows(table, gidx, ncols):
    NR = gidx.shape[0]
    NW = 32
    rows_per_w = NR // NW
    CHUNK = 128
    n_chunks = rows_per_w // CHUNK
    mesh = plsc.VectorSubcoreMesh(core_axis_name="c", subcore_axis_name="s")

    @functools.partial(
        pl.kernel,
        mesh=mesh,
        out_type=jax.ShapeDtypeStruct((NR, ncols), jnp.float32),
        scratch_types=[
            pltpu.VMEM((CHUNK,), jnp.int32),
            pltpu.VMEM((CHUNK, ncols), jnp.float32),
            pltpu.SemaphoreType.DMA,
        ],
        compiler_params=pltpu.CompilerParams(use_tc_tiling_on_sc=False),
    )
    def k(table_hbm, gidx_hbm, out_hbm, idx_v, rows_v, sem):
        wid = lax.axis_index("s") * 2 + lax.axis_index("c")
        base = wid * rows_per_w

        def chunk_body(ci, _):
            cb = base + ci * CHUNK
            pltpu.sync_copy(gidx_hbm.at[pl.ds(cb, CHUNK)], idx_v)
            pltpu.async_copy(table_hbm.at[idx_v], rows_v, sem).wait()
            pltpu.sync_copy(rows_v, out_hbm.at[pl.ds(cb, CHUNK)])
            return 0

        lax.fori_loop(0, n_chunks, chunk_body, 0)

    return k(table, gidx)


NTOT = float(8 * NPOINT * NSAMPLE)
TILE = 512


def _tile_stats(x):
    s = jnp.sum(x, axis=0, keepdims=True)
    sq = jnp.sum(x * x, axis=0, keepdims=True)
    return jnp.concatenate([s, sq], axis=0)


def _acc_stats(ref, x):
    @pl.when(pl.program_id(0) == 0)
    def _():
        ref[...] = jnp.zeros_like(ref)
    ref[...] += _tile_stats(x)


def _bn_consts(stats, g, b):
    mean = stats[0:1] / NTOT
    var = stats[1:2] / NTOT - mean * mean
    a = g / jnp.sqrt(var + EPS)
    c = b - mean * a
    return a, c


def _k0_body(g_ref, nxe_ref, wx_ref, wf_ref, b0_ref, wwx_ref, bw0_ref,
             out_ref, wout_ref, st_ref, wst_ref):
    gt = g_ref[...]
    xn16 = gt[:, 0:16] - nxe_ref[...]
    out0 = (jax.lax.dot_general(xn16, wx_ref[...], (((1,), (0,)), ((), ())),
                                preferred_element_type=jnp.float32)
            + jax.lax.dot_general(gt, wf_ref[...], (((1,), (0,)), ((), ())),
                                  preferred_element_type=jnp.float32)
            + b0_ref[...])
    wout0 = jax.lax.dot_general(xn16, wwx_ref[...], (((1,), (0,)), ((), ())),
                                preferred_element_type=jnp.float32) + bw0_ref[...]
    out_ref[...] = out0
    wout_ref[...] = wout0
    _acc_stats(st_ref, out0)
    _acc_stats(wst_ref, wout0)


def _klayer_body(x_ref, wx_in_ref, st_in_ref, wst_in_ref, g_ref, bb_ref,
                 wg_ref, wbb_ref, w_ref, b_ref, ww_ref, wb_ref,
                 out_ref, wout_ref, st_ref, wst_ref):
    a, c = _bn_consts(st_in_ref[...], g_ref[...], bb_ref[...])
    x = jnp.maximum(x_ref[...] * a + c, 0.0)
    wa, wc = _bn_consts(wst_in_ref[...], wg_ref[...], wbb_ref[...])
    wx = jnp.maximum(wx_in_ref[...] * wa + wc, 0.0)
    out = jax.lax.dot_general(x, w_ref[...], (((1,), (0,)), ((), ())),
                              preferred_element_type=jnp.float32) + b_ref[...]
    wout = jax.lax.dot_general(wx, ww_ref[...], (((1,), (0,)), ((), ())),
                               preferred_element_type=jnp.float32) + wb_ref[...]
    out_ref[...] = out
    wout_ref[...] = wout
    _acc_stats(st_ref, out)
    _acc_stats(wst_ref, wout)


_NB = 64


def _k3_body(x_ref, wx_ref, st_ref, wst_ref, g_ref, bb_ref, wg_ref, wbb_ref, gt_ref):
    a, c = _bn_consts(st_ref[...], g_ref[...], bb_ref[...])
    x = jnp.maximum(x_ref[...] * a + c, 0.0)
    wa, wc = _bn_consts(wst_ref[...], wg_ref[...], wbb_ref[...])
    wx = jnp.maximum(wx_ref[...] * wa + wc, 0.0)
    for i in range(_NB):
        xi = x[i * NSAMPLE:(i + 1) * NSAMPLE]
        wi = wx[i * NSAMPLE:(i + 1) * NSAMPLE]
        gt_ref[:, i, :] = jax.lax.dot_general(
            wi, xi, (((0,), (0,)), ((), ())), preferred_element_type=jnp.float32)


def _k4_body(gt_ref, w3_ref, lb_ref, lin_ref, st_ref, acc_ref):
    b = pl.program_id(0)
    j = pl.program_id(1)

    @pl.when(j == 0)
    def _():
        acc_ref[...] = jnp.broadcast_to(lb_ref[...], acc_ref.shape)

    acc_ref[...] += jax.lax.dot_general(
        gt_ref[0], w3_ref[0], (((1,), (0,)), ((), ())),
        preferred_element_type=jnp.float32)

    @pl.when(j == 15)
    def _():
        a = acc_ref[...]
        lin_ref[...] = a

        @pl.when(b == 0)
        def _():
            st_ref[...] = jnp.zeros_like(st_ref)

        st_ref[...] += _tile_stats(a)


def _k5_body(x_ref, st_ref, g_ref, bb_ref, o_ref):
    a, c = _bn_consts(st_ref[...] * (NTOT / (8.0 * NPOINT)), g_ref[...], bb_ref[...])
    o_ref[...] = jnp.maximum(x_ref[...] * a + c, 0.0)


def _mlp_pallas(G, new_xyz, params):
    R = G.shape[0]
    nsteps = R // TILE

    def pad_rows(m, rows, at, total):
        z0 = jnp.zeros((at, m.shape[1]), jnp.float32)
        z1 = jnp.zeros((total - at - rows, m.shape[1]), jnp.float32)
        return jnp.concatenate([z0, m, z1], axis=0)

    def pad_cols(v, total):
        return jnp.concatenate([v, jnp.zeros((total - v.shape[0],), jnp.float32)])

    w0t = params['conv0_w'].T
    wx16 = pad_rows(w0t[0:3], 3, 0, 16)
    wf144 = pad_rows(w0t[3:131], 128, 3, 144)
    b0 = params['conv0_b'][None, :]
    ww0 = jnp.pad(params['wconv0_w'].T, ((0, 13), (0, 8)))
    bw0 = pad_cols(params['wconv0_b'], 16)[None, :]
    nxe = jnp.repeat(
        jnp.concatenate([new_xyz, jnp.zeros((8, NPOINT, 13), jnp.float32)],
                        axis=2).reshape(8 * NPOINT, 16), NSAMPLE, axis=0)

    row_spec = lambda w: pl.BlockSpec((TILE, w), lambda i: (i, 0))
    full_spec = lambda a: pl.BlockSpec(a.shape, lambda i: tuple(0 for _ in a.shape))
    stat_spec = lambda w: pl.BlockSpec((2, w), lambda i: (0, 0))

    out0, wout0, st0, wst0 = pl.pallas_call(
        _k0_body,
        grid=(nsteps,),
        in_specs=[row_spec(144), row_spec(16)] + [full_spec(a) for a in (wx16, wf144, b0, ww0, bw0)],
        out_specs=(row_spec(128), row_spec(16), stat_spec(128), stat_spec(16)),
        out_shape=(jax.ShapeDtypeStruct((R, 128), jnp.float32),
                   jax.ShapeDtypeStruct((R, 16), jnp.float32),
                   jax.ShapeDtypeStruct((2, 128), jnp.float32),
                   jax.ShapeDtypeStruct((2, 16), jnp.float32)),
    )(G, nxe, wx16, wf144, b0, ww0, bw0)

    def layer(i, x, wx, st, wst, oc):
        ic = x.shape[1]
        g = params['bn%d_g' % (i - 1)][None, :]
        bb = params['bn%d_b' % (i - 1)][None, :]
        wg = pad_cols(params['wbn%d_g' % (i - 1)], 16)[None, :]
        wbb = pad_cols(params['wbn%d_b' % (i - 1)], 16)[None, :]
        w = params['conv%d_w' % i].T
        b = params['conv%d_b' % i][None, :]
        wwt = params['wconv%d_w' % i].T
        ww = jnp.pad(wwt, ((0, 16 - wwt.shape[0]), (0, 16 - wwt.shape[1])))
        wb = pad_cols(params['wconv%d_b' % i], 16)[None, :]
        return pl.pallas_call(
            _klayer_body,
            grid=(nsteps,),
            in_specs=[row_spec(ic), row_spec(16), stat_spec(ic), stat_spec(16)]
                     + [full_spec(a) for a in (g, bb, wg, wbb, w, b, ww, wb)],
            out_specs=(row_spec(oc), row_spec(16), stat_spec(oc), stat_spec(16)),
            out_shape=(jax.ShapeDtypeStruct((R, oc), jnp.float32),
                       jax.ShapeDtypeStruct((R, 16), jnp.float32),
                       jax.ShapeDtypeStruct((2, oc), jnp.float32),
                       jax.ShapeDtypeStruct((2, 16), jnp.float32)),
        )(x, wx, st, wst, g, bb, wg, wbb, w, b, ww, wb)

    out1, wout1, st1, wst1 = layer(1, out0, wout0, st0, wst0, 128)
    out2, wout2, st2, wst2 = layer(2, out1, wout1, st1, wst1, 256)

    g2 = params['bn2_g'][None, :]
    bb2 = params['bn2_b'][None, :]
    wg2 = pad_cols(params['wbn2_g'], 16)[None, :]
    wbb2 = pad_cols(params['wbn2_b'], 16)[None, :]
    n_total = 8 * NPOINT
    gt = pl.pallas_call(
        _k3_body,
        grid=(n_total // _NB,),
        in_specs=[pl.BlockSpec((_NB * NSAMPLE, 256), lambda i: (i, 0)),
                  pl.BlockSpec((_NB * NSAMPLE, 16), lambda i: (i, 0)),
                  stat_spec(256), stat_spec(16),
                  full_spec(g2), full_spec(bb2), full_spec(wg2), full_spec(wbb2)],
        out_specs=pl.BlockSpec((16, _NB, 256), lambda i: (0, i, 0)),
        out_shape=jax.ShapeDtypeStruct((16, n_total, 256), jnp.float32),
    )(out2, wout2, st2, wst2, g2, bb2, wg2, wbb2)

    w3 = params['lin_w'].reshape(256, 256, 16).transpose(2, 1, 0)
    lb = params['lin_b'][None, :]
    lin, stl = pl.pallas_call(
        _k4_body,
        grid=(8, 16),
        in_specs=[pl.BlockSpec((1, NPOINT, 256), lambda b, j: (j, b, 0)),
                  pl.BlockSpec((1, 256, 256), lambda b, j: (j, 0, 0)),
                  pl.BlockSpec((1, 256), lambda b, j: (0, 0))],
        out_specs=(pl.BlockSpec((NPOINT, 256), lambda b, j: (b, 0)),
                   pl.BlockSpec((2, 256), lambda b, j: (0, 0))),
        out_shape=(jax.ShapeDtypeStruct((8 * NPOINT, 256), jnp.float32),
                   jax.ShapeDtypeStruct((2, 256), jnp.float32)),
        scratch_shapes=[pltpu.VMEM((NPOINT, 256), jnp.float32)],
    )(gt, w3, lb)

    gl = params['bnl_g'][None, :]
    bl = params['bnl_b'][None, :]
    out = pl.pallas_call(
        _k5_body,
        grid=(8,),
        in_specs=[pl.BlockSpec((NPOINT, 256), lambda b: (b, 0)),
                  pl.BlockSpec((2, 256), lambda b: (0, 0)),
                  full_spec(gl), full_spec(bl)],
        out_specs=pl.BlockSpec((NPOINT, 256), lambda b: (b, 0)),
        out_shape=jax.ShapeDtypeStruct((8 * NPOINT, 256), jnp.float32),
    )(lin, stl, gl, bl)
    return out.reshape(8, NPOINT, 256)


def kernel(xyz, points, params):
    B = xyz.shape[0]
    xyz_p = xyz.transpose(0, 2, 1)
    pts_p = points.transpose(0, 2, 1)
    fps_idx, new_xyz = _fps_pallas(xyz_p)
    idx = _knn_pallas(xyz_p, new_xyz)
    N = xyz_p.shape[1]
    table = jnp.concatenate(
        [xyz_p, pts_p, jnp.zeros((B, N, 13), jnp.float32)], axis=2).reshape(B * N, 144)
    gidx = (idx + (jnp.arange(B, dtype=jnp.int32) * N)[:, None, None]).reshape(-1)
    G = _sc_gather_rows(table, gidx, 144)
    out = _mlp_pallas(G, new_xyz, params)
    return (new_xyz.transpose(0, 2, 1), out.transpose(0, 2, 1))


# speedup vs baseline: 1.0640x; 1.0322x over previous
import functools

import jax
import jax.numpy as jnp
import numpy as np
from jax import lax
from jax.experimental import pallas as pl
from jax.experimental.pallas import tpu as pltpu
from jax.experimental.pallas import tpu_sc as plsc

EPS = 1e-5
NPOINT = 512
NSAMPLE = 32


def _index_points(points, idx):
    return jax.vmap(lambda p, i: p[i])(points, idx)


def _fps_kernel(x_ref, y_ref, z_ref, far0_ref, idx_ref, cx_ref, cy_ref, cz_ref):
    B, N = x_ref.shape
    iota_l = jax.lax.broadcasted_iota(jnp.int32, (B, N), 1)
    iota_p = jax.lax.broadcasted_iota(jnp.int32, (B, NPOINT), 1)
    x = x_ref[...]
    y = y_ref[...]
    z = z_ref[...]

    def body(i, st):
        dist, far, idx_acc, cx_acc, cy_acc, cz_acc = st
        mask = iota_l == far
        cx = jnp.sum(jnp.where(mask, x, 0.0), axis=1, keepdims=True)
        cy = jnp.sum(jnp.where(mask, y, 0.0), axis=1, keepdims=True)
        cz = jnp.sum(jnp.where(mask, z, 0.0), axis=1, keepdims=True)
        sel = iota_p == i
        idx_acc = jnp.where(sel, far, idx_acc)
        cx_acc = jnp.where(sel, cx, cx_acc)
        cy_acc = jnp.where(sel, cy, cy_acc)
        cz_acc = jnp.where(sel, cz, cz_acc)
        dx = x - cx
        dy = y - cy
        dz = z - cz
        d = (dx * dx + dy * dy) + dz * dz
        dist = jnp.minimum(dist, d)
        m = jnp.max(dist, axis=1, keepdims=True)
        far = jnp.min(jnp.where(dist == m, iota_l, N), axis=1, keepdims=True)
        return (dist, far, idx_acc, cx_acc, cy_acc, cz_acc)

    dist0 = jnp.full((B, N), 1e10, dtype=jnp.float32)
    zp = jnp.zeros((B, NPOINT), dtype=jnp.float32)
    zi = jnp.zeros((B, NPOINT), dtype=jnp.int32)
    _, _, idx_acc, cx_acc, cy_acc, cz_acc = jax.lax.fori_loop(
        0, NPOINT, body, (dist0, far0_ref[...], zi, zp, zp, zp))
    idx_ref[...] = idx_acc
    cx_ref[...] = cx_acc
    cy_ref[...] = cy_acc
    cz_ref[...] = cz_acc


def _fps_pallas(xyz_p):
    B, N, _ = xyz_p.shape
    far0 = jax.random.randint(jax.random.key(42), (B,), 0, N).astype(jnp.int32)[:, None]
    x = xyz_p[:, :, 0]
    y = xyz_p[:, :, 1]
    z = xyz_p[:, :, 2]
    idx, cx, cy, cz = pl.pallas_call(
        _fps_kernel,
        out_shape=(
            jax.ShapeDtypeStruct((B, NPOINT), jnp.int32),
            jax.ShapeDtypeStruct((B, NPOINT), jnp.float32),
            jax.ShapeDtypeStruct((B, NPOINT), jnp.float32),
            jax.ShapeDtypeStruct((B, NPOINT), jnp.float32),
        ),
    )(x, y, z, far0)
    new_xyz = jnp.stack([cx, cy, cz], axis=2)
    return idx, new_xyz


def _knn_kernel(nx8_ref, xyzT8_ref, idx_ref):
    M, N = 512, 4096
    nx8 = nx8_ref[0]
    xyzT8 = xyzT8_ref[0]
    mm = jax.lax.dot_general(nx8, xyzT8, (((1,), (0,)), ((), ())),
                             preferred_element_type=jnp.float32)
    sqr = -2.0 * mm
    sqr = sqr + jnp.sum(nx8 * nx8, axis=1, keepdims=True)
    sqr = sqr + jnp.sum(xyzT8 * xyzT8, axis=0, keepdims=True)
    CH, L = 32, 128
    INF = jnp.float32(jnp.inf)
    iota_l = jax.lax.broadcasted_iota(jnp.int32, (M, N), 1)
    lane_i = jax.lax.broadcasted_iota(jnp.int32, (M, L), 1)
    mins = [jnp.min(sqr[:, c * L:(c + 1) * L], axis=1, keepdims=True) for c in range(CH)]
    Mm = jnp.concatenate(mins, axis=1)
    ch_iota = jax.lax.broadcasted_iota(jnp.int32, (M, CH), 1)
    cols = []
    for _ in range(NSAMPLE):
        cM = jnp.argmin(Mm, axis=1).astype(jnp.int32)[:, None]
        Y = jnp.zeros((M, L), jnp.float32)
        for c in range(CH):
            Y = Y + jnp.where(cM == c, sqr[:, c * L:(c + 1) * L], 0.0)
        lstar = jnp.argmin(Y, axis=1).astype(jnp.int32)[:, None]
        sel = cM * L + lstar
        cols.append(sel)
        sqr = jnp.where(iota_l == sel, INF, sqr)
        newmin = jnp.min(jnp.where(lane_i == lstar, INF, Y), axis=1, keepdims=True)
        Mm = jnp.where(ch_iota == cM, newmin, Mm)
    idx_ref[0] = jnp.concatenate(cols, axis=1)


def _knn_pallas(xyz_p, new_xyz):
    B, N, _ = xyz_p.shape
    nx8 = jnp.concatenate([new_xyz, jnp.zeros((B, NPOINT, 5), jnp.float32)], axis=2)
    xyzT8 = jnp.concatenate([xyz_p.transpose(0, 2, 1), jnp.zeros((B, 5, N), jnp.float32)], axis=1)
    idx = pl.pallas_call(
        _knn_kernel,
        grid=(B,),
        in_specs=[
            pl.BlockSpec((1, NPOINT, 8), lambda b: (b, 0, 0)),
            pl.BlockSpec((1, 8, N), lambda b: (b, 0, 0)),
        ],
        out_specs=pl.BlockSpec((1, NPOINT, NSAMPLE), lambda b: (b, 0, 0)),
        out_shape=jax.ShapeDtypeStruct((B, NPOINT, NSAMPLE), jnp.int32),
    )(nx8, xyzT8)
    return idx


def _sc_gather_rows(table, gidx, ncols):
    NR = gidx.shape[0]
    NW = 32
    rows_per_w = NR // NW
    CHUNK = 128
    n_chunks = rows_per_w // CHUNK
    mesh = plsc.VectorSubcoreMesh(core_axis_name="c", subcore_axis_name="s")

    @functools.partial(
        pl.kernel,
        mesh=mesh,
        out_type=jax.ShapeDtypeStruct((NR, ncols), jnp.float32),
        scratch_types=[
            pltpu.VMEM((CHUNK,), jnp.int32),
            pltpu.VMEM((CHUNK, ncols), jnp.float32),
            pltpu.SemaphoreType.DMA,
        ],
        compiler_params=pltpu.CompilerParams(use_tc_tiling_on_sc=False),
    )
    def k(table_hbm, gidx_hbm, out_hbm, idx_v, rows_v, sem):
        wid = lax.axis_index("s") * 2 + lax.axis_index("c")
        base = wid * rows_per_w

        def chunk_body(ci, _):
            cb = base + ci * CHUNK
            pltpu.sync_copy(gidx_hbm.at[pl.ds(cb, CHUNK)], idx_v)
            pltpu.async_copy(table_hbm.at[idx_v], rows_v, sem).wait()
            pltpu.sync_copy(rows_v, out_hbm.at[pl.ds(cb, CHUNK)])
            return 0

        lax.fori_loop(0, n_chunks, chunk_body, 0)

    return k(table, gidx)


NTOT = float(8 * NPOINT * NSAMPLE)
TILE = 512


def _tile_stats(x):
    s = jnp.sum(x, axis=0, keepdims=True)
    sq = jnp.sum(x * x, axis=0, keepdims=True)
    return jnp.concatenate([s, sq], axis=0)


def _acc_stats(ref, x):
    @pl.when(pl.program_id(0) == 0)
    def _():
        ref[...] = jnp.zeros_like(ref)
    ref[...] += _tile_stats(x)


def _bn_consts(stats, g, b):
    mean = stats[0:1] / NTOT
    var = stats[1:2] / NTOT - mean * mean
    a = g / jnp.sqrt(var + EPS)
    c = b - mean * a
    return a, c


def _k0_body(g_ref, nxe_ref, wx_ref, wf_ref, b0_ref, wwx_ref, bw0_ref,
             out_ref, wout_ref, st_ref, wst_ref):
    gt = g_ref[...]
    xn16 = gt[:, 0:16] - nxe_ref[...]
    out0 = (jax.lax.dot_general(xn16, wx_ref[...], (((1,), (0,)), ((), ())),
                                preferred_element_type=jnp.float32)
            + jax.lax.dot_general(gt, wf_ref[...], (((1,), (0,)), ((), ())),
                                  preferred_element_type=jnp.float32)
            + b0_ref[...])
    wout0 = jax.lax.dot_general(xn16, wwx_ref[...], (((1,), (0,)), ((), ())),
                                preferred_element_type=jnp.float32) + bw0_ref[...]
    out_ref[...] = out0
    wout_ref[...] = wout0
    _acc_stats(st_ref, out0)
    _acc_stats(wst_ref, wout0)


def _klayer_body(x_ref, wx_in_ref, st_in_ref, wst_in_ref, g_ref, bb_ref,
                 wg_ref, wbb_ref, w_ref, b_ref, ww_ref, wb_ref,
                 out_ref, wout_ref, st_ref, wst_ref):
    a, c = _bn_consts(st_in_ref[...], g_ref[...], bb_ref[...])
    x = jnp.maximum(x_ref[...] * a + c, 0.0)
    wa, wc = _bn_consts(wst_in_ref[...], wg_ref[...], wbb_ref[...])
    wx = jnp.maximum(wx_in_ref[...] * wa + wc, 0.0)
    out = jax.lax.dot_general(x, w_ref[...], (((1,), (0,)), ((), ())),
                              preferred_element_type=jnp.float32) + b_ref[...]
    wout = jax.lax.dot_general(wx, ww_ref[...], (((1,), (0,)), ((), ())),
                               preferred_element_type=jnp.float32) + wb_ref[...]
    out_ref[...] = out
    wout_ref[...] = wout
    _acc_stats(st_ref, out)
    _acc_stats(wst_ref, wout)


_NB = 64


def _k3_body(x_ref, wx_ref, st_ref, wst_ref, g_ref, bb_ref, wg_ref, wbb_ref, gt_ref):
    a, c = _bn_consts(st_ref[...], g_ref[...], bb_ref[...])
    x = jnp.maximum(x_ref[...] * a + c, 0.0)
    wa, wc = _bn_consts(wst_ref[...], wg_ref[...], wbb_ref[...])
    wx = jnp.maximum(wx_ref[...] * wa + wc, 0.0)
    for i in range(_NB):
        xi = x[i * NSAMPLE:(i + 1) * NSAMPLE]
        wi = wx[i * NSAMPLE:(i + 1) * NSAMPLE]
        gt_ref[:, i, :] = jax.lax.dot_general(
            wi, xi, (((0,), (0,)), ((), ())), preferred_element_type=jnp.float32)


def _k4_body(gt_ref, w3_ref, lb_ref, lin_ref, st_ref, acc_ref):
    b = pl.program_id(0)
    j = pl.program_id(1)

    @pl.when(j == 0)
    def _():
        acc_ref[...] = jnp.broadcast_to(lb_ref[...], acc_ref.shape)

    acc_ref[...] += jax.lax.dot_general(
        gt_ref[0], w3_ref[0], (((1,), (0,)), ((), ())),
        preferred_element_type=jnp.float32)

    @pl.when(j == 15)
    def _():
        a = acc_ref[...]
        lin_ref[...] = a

        @pl.when(b == 0)
        def _():
            st_ref[...] = jnp.zeros_like(st_ref)

        st_ref[...] += _tile_stats(a)


def _k5_body(x_ref, st_ref, g_ref, bb_ref, o_ref):
    a, c = _bn_consts(st_ref[...] * (NTOT / (8.0 * NPOINT)), g_ref[...], bb_ref[...])
    o_ref[...] = jnp.maximum(x_ref[...] * a + c, 0.0)


def _mlp_pallas(G, new_xyz, params):
    R = G.shape[0]
    nsteps = R // TILE

    def pad_rows(m, rows, at, total):
        z0 = jnp.zeros((at, m.shape[1]), jnp.float32)
        z1 = jnp.zeros((total - at - rows, m.shape[1]), jnp.float32)
        return jnp.concatenate([z0, m, z1], axis=0)

    def pad_cols(v, total):
        return jnp.concatenate([v, jnp.zeros((total - v.shape[0],), jnp.float32)])

    w0t = params['conv0_w'].T
    wx16 = pad_rows(w0t[0:3], 3, 0, 16)
    wf144 = pad_rows(w0t[3:131], 128, 3, 144)
    b0 = params['conv0_b'][None, :]
    ww0 = jnp.pad(params['wconv0_w'].T, ((0, 13), (0, 8)))
    bw0 = pad_cols(params['wconv0_b'], 16)[None, :]
    nxe = jnp.repeat(
        jnp.concatenate([new_xyz, jnp.zeros((8, NPOINT, 13), jnp.float32)],
                        axis=2).reshape(8 * NPOINT, 16), NSAMPLE, axis=0)

    row_spec = lambda w: pl.BlockSpec((TILE, w), lambda i: (i, 0))
    full_spec = lambda a: pl.BlockSpec(a.shape, lambda i: tuple(0 for _ in a.shape))
    stat_spec = lambda w: pl.BlockSpec((2, w), lambda i: (0, 0))

    out0, wout0, st0, wst0 = pl.pallas_call(
        _k0_body,
        grid=(nsteps,),
        in_specs=[row_spec(144), row_spec(16)] + [full_spec(a) for a in (wx16, wf144, b0, ww0, bw0)],
        out_specs=(row_spec(128), row_spec(16), stat_spec(128), stat_spec(16)),
        out_shape=(jax.ShapeDtypeStruct((R, 128), jnp.float32),
                   jax.ShapeDtypeStruct((R, 16), jnp.float32),
                   jax.ShapeDtypeStruct((2, 128), jnp.float32),
                   jax.ShapeDtypeStruct((2, 16), jnp.float32)),
    )(G, nxe, wx16, wf144, b0, ww0, bw0)

    def layer(i, x, wx, st, wst, oc):
        ic = x.shape[1]
        g = params['bn%d_g' % (i - 1)][None, :]
        bb = params['bn%d_b' % (i - 1)][None, :]
        wg = pad_cols(params['wbn%d_g' % (i - 1)], 16)[None, :]
        wbb = pad_cols(params['wbn%d_b' % (i - 1)], 16)[None, :]
        w = params['conv%d_w' % i].T
        b = params['conv%d_b' % i][None, :]
        wwt = params['wconv%d_w' % i].T
        ww = jnp.pad(wwt, ((0, 16 - wwt.shape[0]), (0, 16 - wwt.shape[1])))
        wb = pad_cols(params['wconv%d_b' % i], 16)[None, :]
        return pl.pallas_call(
            _klayer_body,
            grid=(nsteps,),
            in_specs=[row_spec(ic), row_spec(16), stat_spec(ic), stat_spec(16)]
                     + [full_spec(a) for a in (g, bb, wg, wbb, w, b, ww, wb)],
            out_specs=(row_spec(oc), row_spec(16), stat_spec(oc), stat_spec(16)),
            out_shape=(jax.ShapeDtypeStruct((R, oc), jnp.float32),
                       jax.ShapeDtypeStruct((R, 16), jnp.float32),
                       jax.ShapeDtypeStruct((2, oc), jnp.float32),
                       jax.ShapeDtypeStruct((2, 16), jnp.float32)),
        )(x, wx, st, wst, g, bb, wg, wbb, w, b, ww, wb)

    out1, wout1, st1, wst1 = layer(1, out0, wout0, st0, wst0, 128)
    out2, wout2, st2, wst2 = layer(2, out1, wout1, st1, wst1, 256)

    g2 = params['bn2_g'][None, :]
    bb2 = params['bn2_b'][None, :]
    wg2 = pad_cols(params['wbn2_g'], 16)[None, :]
    wbb2 = pad_cols(params['wbn2_b'], 16)[None, :]
    n_total = 8 * NPOINT
    gt = pl.pallas_call(
        _k3_body,
        grid=(n_total // _NB,),
        in_specs=[pl.BlockSpec((_NB * NSAMPLE, 256), lambda i: (i, 0)),
                  pl.BlockSpec((_NB * NSAMPLE, 16), lambda i: (i, 0)),
                  stat_spec(256), stat_spec(16),
                  full_spec(g2), full_spec(bb2), full_spec(wg2), full_spec(wbb2)],
        out_specs=pl.BlockSpec((16, _NB, 256), lambda i: (0, i, 0)),
        out_shape=jax.ShapeDtypeStruct((16, n_total, 256), jnp.float32),
    )(out2, wout2, st2, wst2, g2, bb2, wg2, wbb2)

    w3 = params['lin_w'].reshape(256, 256, 16).transpose(2, 1, 0)
    lb = params['lin_b'][None, :]
    lin, stl = pl.pallas_call(
        _k4_body,
        grid=(8, 16),
        in_specs=[pl.BlockSpec((1, NPOINT, 256), lambda b, j: (j, b, 0)),
                  pl.BlockSpec((1, 256, 256), lambda b, j: (j, 0, 0)),
                  pl.BlockSpec((1, 256), lambda b, j: (0, 0))],
        out_specs=(pl.BlockSpec((NPOINT, 256), lambda b, j: (b, 0)),
                   pl.BlockSpec((2, 256), lambda b, j: (0, 0))),
        out_shape=(jax.ShapeDtypeStruct((8 * NPOINT, 256), jnp.float32),
                   jax.ShapeDtypeStruct((2, 256), jnp.float32)),
        scratch_shapes=[pltpu.VMEM((NPOINT, 256), jnp.float32)],
    )(gt, w3, lb)

    gl = params['bnl_g'][None, :]
    bl = params['bnl_b'][None, :]
    out = pl.pallas_call(
        _k5_body,
        grid=(8,),
        in_specs=[pl.BlockSpec((NPOINT, 256), lambda b: (b, 0)),
                  pl.BlockSpec((2, 256), lambda b: (0, 0)),
                  full_spec(gl), full_spec(bl)],
        out_specs=pl.BlockSpec((NPOINT, 256), lambda b: (b, 0)),
        out_shape=jax.ShapeDtypeStruct((8 * NPOINT, 256), jnp.float32),
    )(lin, stl, gl, bl)
    return out.reshape(8, NPOINT, 256)


def kernel(xyz, points, params):
    B = xyz.shape[0]
    xyz_p = xyz.transpose(0, 2, 1)
    pts_p = points.transpose(0, 2, 1)
    fps_idx, new_xyz = _fps_pallas(xyz_p)
    idx = _knn_pallas(xyz_p, new_xyz)
    N = xyz_p.shape[1]
    table = jnp.concatenate(
        [xyz_p, pts_p, jnp.zeros((B, N, 13), jnp.float32)], axis=2).reshape(B * N, 144)
    gidx = (idx + (jnp.arange(B, dtype=jnp.int32) * N)[:, None, None]).reshape(-1)
    G = _sc_gather_rows(table, gidx, 144)
    out = _mlp_pallas(G, new_xyz, params)
    return (new_xyz.transpose(0, 2, 1), out.transpose(0, 2, 1))


# speedup vs baseline: 1.0785x; 1.0136x over previous
import functools

import jax
import jax.numpy as jnp
import numpy as np
from jax import lax
from jax.experimental import pallas as pl
from jax.experimental.pallas import tpu as pltpu
from jax.experimental.pallas import tpu_sc as plsc

EPS = 1e-5
NPOINT = 512
NSAMPLE = 32


def _index_points(points, idx):
    return jax.vmap(lambda p, i: p[i])(points, idx)


def _fps_kernel(x_ref, y_ref, z_ref, far0_ref, idx_ref, cx_ref, cy_ref, cz_ref):
    B, N = x_ref.shape
    iota_l = jax.lax.broadcasted_iota(jnp.int32, (B, N), 1)
    iota_p = jax.lax.broadcasted_iota(jnp.int32, (B, NPOINT), 1)
    x = x_ref[...]
    y = y_ref[...]
    z = z_ref[...]

    def body(i, st):
        dist, far, idx_acc, cx_acc, cy_acc, cz_acc = st
        mask = iota_l == far
        cx = jnp.sum(jnp.where(mask, x, 0.0), axis=1, keepdims=True)
        cy = jnp.sum(jnp.where(mask, y, 0.0), axis=1, keepdims=True)
        cz = jnp.sum(jnp.where(mask, z, 0.0), axis=1, keepdims=True)
        sel = iota_p == i
        idx_acc = jnp.where(sel, far, idx_acc)
        cx_acc = jnp.where(sel, cx, cx_acc)
        cy_acc = jnp.where(sel, cy, cy_acc)
        cz_acc = jnp.where(sel, cz, cz_acc)
        dx = x - cx
        dy = y - cy
        dz = z - cz
        d = (dx * dx + dy * dy) + dz * dz
        dist = jnp.minimum(dist, d)
        m = jnp.max(dist, axis=1, keepdims=True)
        far = jnp.min(jnp.where(dist == m, iota_l, N), axis=1, keepdims=True)
        return (dist, far, idx_acc, cx_acc, cy_acc, cz_acc)

    dist0 = jnp.full((B, N), 1e10, dtype=jnp.float32)
    zp = jnp.zeros((B, NPOINT), dtype=jnp.float32)
    zi = jnp.zeros((B, NPOINT), dtype=jnp.int32)
    _, _, idx_acc, cx_acc, cy_acc, cz_acc = jax.lax.fori_loop(
        0, NPOINT, body, (dist0, far0_ref[...], zi, zp, zp, zp))
    idx_ref[...] = idx_acc
    cx_ref[...] = cx_acc
    cy_ref[...] = cy_acc
    cz_ref[...] = cz_acc


def _fps_pallas(xyz_p):
    B, N, _ = xyz_p.shape
    far0 = jax.random.randint(jax.random.key(42), (B,), 0, N).astype(jnp.int32)[:, None]
    x = xyz_p[:, :, 0]
    y = xyz_p[:, :, 1]
    z = xyz_p[:, :, 2]
    idx, cx, cy, cz = pl.pallas_call(
        _fps_kernel,
        out_shape=(
            jax.ShapeDtypeStruct((B, NPOINT), jnp.int32),
            jax.ShapeDtypeStruct((B, NPOINT), jnp.float32),
            jax.ShapeDtypeStruct((B, NPOINT), jnp.float32),
            jax.ShapeDtypeStruct((B, NPOINT), jnp.float32),
        ),
    )(x, y, z, far0)
    new_xyz = jnp.stack([cx, cy, cz], axis=2)
    return idx, new_xyz


def _knn_kernel(nx8_ref, xyzT8_ref, idx_ref):
    M, N = 512, 4096
    nx8 = nx8_ref[0]
    xyzT8 = xyzT8_ref[0]
    mm = jax.lax.dot_general(nx8, xyzT8, (((1,), (0,)), ((), ())),
                             preferred_element_type=jnp.float32)
    sqr = -2.0 * mm
    sqr = sqr + jnp.sum(nx8 * nx8, axis=1, keepdims=True)
    sqr = sqr + jnp.sum(xyzT8 * xyzT8, axis=0, keepdims=True)
    CH, L = 32, 128
    INF = jnp.float32(jnp.inf)
    iota_l = jax.lax.broadcasted_iota(jnp.int32, (M, N), 1)
    lane_i = jax.lax.broadcasted_iota(jnp.int32, (M, L), 1)
    mins = [jnp.min(sqr[:, c * L:(c + 1) * L], axis=1, keepdims=True) for c in range(CH)]
    Mm = jnp.concatenate(mins, axis=1)
    ch_iota = jax.lax.broadcasted_iota(jnp.int32, (M, CH), 1)
    cols = []
    for _ in range(NSAMPLE):
        cM = jnp.argmin(Mm, axis=1).astype(jnp.int32)[:, None]
        Y = jnp.zeros((M, L), jnp.float32)
        for c in range(CH):
            Y = Y + jnp.where(cM == c, sqr[:, c * L:(c + 1) * L], 0.0)
        for e in cols:
            Y = jnp.where(lane_i == e - cM * L, INF, Y)
        lstar = jnp.argmin(Y, axis=1).astype(jnp.int32)[:, None]
        sel = cM * L + lstar
        cols.append(sel)
        newmin = jnp.min(jnp.where(lane_i == lstar, INF, Y), axis=1, keepdims=True)
        Mm = jnp.where(ch_iota == cM, newmin, Mm)
    idx_ref[0] = jnp.concatenate(cols, axis=1)


def _knn_pallas(xyz_p, new_xyz):
    B, N, _ = xyz_p.shape
    nx8 = jnp.concatenate([new_xyz, jnp.zeros((B, NPOINT, 5), jnp.float32)], axis=2)
    xyzT8 = jnp.concatenate([xyz_p.transpose(0, 2, 1), jnp.zeros((B, 5, N), jnp.float32)], axis=1)
    idx = pl.pallas_call(
        _knn_kernel,
        grid=(B,),
        in_specs=[
            pl.BlockSpec((1, NPOINT, 8), lambda b: (b, 0, 0)),
            pl.BlockSpec((1, 8, N), lambda b: (b, 0, 0)),
        ],
        out_specs=pl.BlockSpec((1, NPOINT, NSAMPLE), lambda b: (b, 0, 0)),
        out_shape=jax.ShapeDtypeStruct((B, NPOINT, NSAMPLE), jnp.int32),
    )(nx8, xyzT8)
    return idx


def _sc_gather_rows(table, gidx, ncols):
    NR = gidx.shape[0]
    NW = 32
    rows_per_w = NR // NW
    CHUNK = 128
    n_chunks = rows_per_w // CHUNK
    mesh = plsc.VectorSubcoreMesh(core_axis_name="c", subcore_axis_name="s")

    @functools.partial(
        pl.kernel,
        mesh=mesh,
        out_type=jax.ShapeDtypeStruct((NR, ncols), jnp.float32),
        scratch_types=[
            pltpu.VMEM((CHUNK,), jnp.int32),
            pltpu.VMEM((CHUNK, ncols), jnp.float32),
            pltpu.SemaphoreType.DMA,
        ],
        compiler_params=pltpu.CompilerParams(use_tc_tiling_on_sc=False),
    )
    def k(table_hbm, gidx_hbm, out_hbm, idx_v, rows_v, sem):
        wid = lax.axis_index("s") * 2 + lax.axis_index("c")
        base = wid * rows_per_w

        def chunk_body(ci, _):
            cb = base + ci * CHUNK
            pltpu.sync_copy(gidx_hbm.at[pl.ds(cb, CHUNK)], idx_v)
            pltpu.async_copy(table_hbm.at[idx_v], rows_v, sem).wait()
            pltpu.sync_copy(rows_v, out_hbm.at[pl.ds(cb, CHUNK)])
            return 0

        lax.fori_loop(0, n_chunks, chunk_body, 0)

    return k(table, gidx)


NTOT = float(8 * NPOINT * NSAMPLE)
TILE = 512


def _tile_stats(x):
    s = jnp.sum(x, axis=0, keepdims=True)
    sq = jnp.sum(x * x, axis=0, keepdims=True)
    return jnp.concatenate([s, sq], axis=0)


def _acc_stats(ref, x):
    @pl.when(pl.program_id(0) == 0)
    def _():
        ref[...] = jnp.zeros_like(ref)
    ref[...] += _tile_stats(x)


def _bn_consts(stats, g, b):
    mean = stats[0:1] / NTOT
    var = stats[1:2] / NTOT - mean * mean
    a = g / jnp.sqrt(var + EPS)
    c = b - mean * a
    return a, c


def _k0_body(g_ref, nxe_ref, wx_ref, wf_ref, b0_ref, wwx_ref, bw0_ref,
             out_ref, wout_ref, st_ref, wst_ref):
    gt = g_ref[...]
    xn16 = gt[:, 0:16] - nxe_ref[...]
    out0 = (jax.lax.dot_general(xn16, wx_ref[...], (((1,), (0,)), ((), ())),
                                preferred_element_type=jnp.float32)
            + jax.lax.dot_general(gt, wf_ref[...], (((1,), (0,)), ((), ())),
                                  preferred_element_type=jnp.float32)
            + b0_ref[...])
    wout0 = jax.lax.dot_general(xn16, wwx_ref[...], (((1,), (0,)), ((), ())),
                                preferred_element_type=jnp.float32) + bw0_ref[...]
    out_ref[...] = out0
    wout_ref[...] = wout0
    _acc_stats(st_ref, out0)
    _acc_stats(wst_ref, wout0)


def _klayer_body(x_ref, wx_in_ref, st_in_ref, wst_in_ref, g_ref, bb_ref,
                 wg_ref, wbb_ref, w_ref, b_ref, ww_ref, wb_ref,
                 out_ref, wout_ref, st_ref, wst_ref):
    a, c = _bn_consts(st_in_ref[...], g_ref[...], bb_ref[...])
    x = jnp.maximum(x_ref[...] * a + c, 0.0)
    wa, wc = _bn_consts(wst_in_ref[...], wg_ref[...], wbb_ref[...])
    wx = jnp.maximum(wx_in_ref[...] * wa + wc, 0.0)
    out = jax.lax.dot_general(x, w_ref[...], (((1,), (0,)), ((), ())),
                              preferred_element_type=jnp.float32) + b_ref[...]
    wout = jax.lax.dot_general(wx, ww_ref[...], (((1,), (0,)), ((), ())),
                               preferred_element_type=jnp.float32) + wb_ref[...]
    out_ref[...] = out
    wout_ref[...] = wout
    _acc_stats(st_ref, out)
    _acc_stats(wst_ref, wout)


_NB = 64


def _k3_body(x_ref, wx_ref, st_ref, wst_ref, g_ref, bb_ref, wg_ref, wbb_ref, gt_ref):
    a, c = _bn_consts(st_ref[...], g_ref[...], bb_ref[...])
    x = jnp.maximum(x_ref[...] * a + c, 0.0)
    wa, wc = _bn_consts(wst_ref[...], wg_ref[...], wbb_ref[...])
    wx = jnp.maximum(wx_ref[...] * wa + wc, 0.0)
    for i in range(_NB):
        xi = x[i * NSAMPLE:(i + 1) * NSAMPLE]
        wi = wx[i * NSAMPLE:(i + 1) * NSAMPLE]
        gt_ref[:, i, :] = jax.lax.dot_general(
            wi, xi, (((0,), (0,)), ((), ())), preferred_element_type=jnp.float32)


def _k4_body(gt_ref, w3_ref, lb_ref, lin_ref, st_ref, acc_ref):
    b = pl.program_id(0)
    j = pl.program_id(1)

    @pl.when(j == 0)
    def _():
        acc_ref[...] = jnp.broadcast_to(lb_ref[...], acc_ref.shape)

    acc_ref[...] += jax.lax.dot_general(
        gt_ref[0], w3_ref[0], (((1,), (0,)), ((), ())),
        preferred_element_type=jnp.float32)

    @pl.when(j == 15)
    def _():
        a = acc_ref[...]
        lin_ref[...] = a

        @pl.when(b == 0)
        def _():
            st_ref[...] = jnp.zeros_like(st_ref)

        st_ref[...] += _tile_stats(a)


def _k5_body(x_ref, st_ref, g_ref, bb_ref, o_ref):
    a, c = _bn_consts(st_ref[...] * (NTOT / (8.0 * NPOINT)), g_ref[...], bb_ref[...])
    o_ref[...] = jnp.maximum(x_ref[...] * a + c, 0.0)


def _mlp_pallas(G, new_xyz, params):
    R = G.shape[0]
    nsteps = R // TILE

    def pad_rows(m, rows, at, total):
        z0 = jnp.zeros((at, m.shape[1]), jnp.float32)
        z1 = jnp.zeros((total - at - rows, m.shape[1]), jnp.float32)
        return jnp.concatenate([z0, m, z1], axis=0)

    def pad_cols(v, total):
        return jnp.concatenate([v, jnp.zeros((total - v.shape[0],), jnp.float32)])

    w0t = params['conv0_w'].T
    wx16 = pad_rows(w0t[0:3], 3, 0, 16)
    wf144 = pad_rows(w0t[3:131], 128, 3, 144)
    b0 = params['conv0_b'][None, :]
    ww0 = jnp.pad(params['wconv0_w'].T, ((0, 13), (0, 8)))
    bw0 = pad_cols(params['wconv0_b'], 16)[None, :]
    nxe = jnp.repeat(
        jnp.concatenate([new_xyz, jnp.zeros((8, NPOINT, 13), jnp.float32)],
                        axis=2).reshape(8 * NPOINT, 16), NSAMPLE, axis=0)

    row_spec = lambda w: pl.BlockSpec((TILE, w), lambda i: (i, 0))
    full_spec = lambda a: pl.BlockSpec(a.shape, lambda i: tuple(0 for _ in a.shape))
    stat_spec = lambda w: pl.BlockSpec((2, w), lambda i: (0, 0))

    out0, wout0, st0, wst0 = pl.pallas_call(
        _k0_body,
        grid=(nsteps,),
        in_specs=[row_spec(144), row_spec(16)] + [full_spec(a) for a in (wx16, wf144, b0, ww0, bw0)],
        out_specs=(row_spec(128), row_spec(16), stat_spec(128), stat_spec(16)),
        out_shape=(jax.ShapeDtypeStruct((R, 128), jnp.float32),
                   jax.ShapeDtypeStruct((R, 16), jnp.float32),
                   jax.ShapeDtypeStruct((2, 128), jnp.float32),
                   jax.ShapeDtypeStruct((2, 16), jnp.float32)),
    )(G, nxe, wx16, wf144, b0, ww0, bw0)

    def layer(i, x, wx, st, wst, oc):
        ic = x.shape[1]
        g = params['bn%d_g' % (i - 1)][None, :]
        bb = params['bn%d_b' % (i - 1)][None, :]
        wg = pad_cols(params['wbn%d_g' % (i - 1)], 16)[None, :]
        wbb = pad_cols(params['wbn%d_b' % (i - 1)], 16)[None, :]
        w = params['conv%d_w' % i].T
        b = params['conv%d_b' % i][None, :]
        wwt = params['wconv%d_w' % i].T
        ww = jnp.pad(wwt, ((0, 16 - wwt.shape[0]), (0, 16 - wwt.shape[1])))
        wb = pad_cols(params['wconv%d_b' % i], 16)[None, :]
        return pl.pallas_call(
            _klayer_body,
            grid=(nsteps,),
            in_specs=[row_spec(ic), row_spec(16), stat_spec(ic), stat_spec(16)]
                     + [full_spec(a) for a in (g, bb, wg, wbb, w, b, ww, wb)],
            out_specs=(row_spec(oc), row_spec(16), stat_spec(oc), stat_spec(16)),
            out_shape=(jax.ShapeDtypeStruct((R, oc), jnp.float32),
                       jax.ShapeDtypeStruct((R, 16), jnp.float32),
                       jax.ShapeDtypeStruct((2, oc), jnp.float32),
                       jax.ShapeDtypeStruct((2, 16), jnp.float32)),
        )(x, wx, st, wst, g, bb, wg, wbb, w, b, ww, wb)

    out1, wout1, st1, wst1 = layer(1, out0, wout0, st0, wst0, 128)
    out2, wout2, st2, wst2 = layer(2, out1, wout1, st1, wst1, 256)

    g2 = params['bn2_g'][None, :]
    bb2 = params['bn2_b'][None, :]
    wg2 = pad_cols(params['wbn2_g'], 16)[None, :]
    wbb2 = pad_cols(params['wbn2_b'], 16)[None, :]
    n_total = 8 * NPOINT
    gt = pl.pallas_call(
        _k3_body,
        grid=(n_total // _NB,),
        in_specs=[pl.BlockSpec((_NB * NSAMPLE, 256), lambda i: (i, 0)),
                  pl.BlockSpec((_NB * NSAMPLE, 16), lambda i: (i, 0)),
                  stat_spec(256), stat_spec(16),
                  full_spec(g2), full_spec(bb2), full_spec(wg2), full_spec(wbb2)],
        out_specs=pl.BlockSpec((16, _NB, 256), lambda i: (0, i, 0)),
        out_shape=jax.ShapeDtypeStruct((16, n_total, 256), jnp.float32),
    )(out2, wout2, st2, wst2, g2, bb2, wg2, wbb2)

    w3 = params['lin_w'].reshape(256, 256, 16).transpose(2, 1, 0)
    lb = params['lin_b'][None, :]
    lin, stl = pl.pallas_call(
        _k4_body,
        grid=(8, 16),
        in_specs=[pl.BlockSpec((1, NPOINT, 256), lambda b, j: (j, b, 0)),
                  pl.BlockSpec((1, 256, 256), lambda b, j: (j, 0, 0)),
                  pl.BlockSpec((1, 256), lambda b, j: (0, 0))],
        out_specs=(pl.BlockSpec((NPOINT, 256), lambda b, j: (b, 0)),
                   pl.BlockSpec((2, 256), lambda b, j: (0, 0))),
        out_shape=(jax.ShapeDtypeStruct((8 * NPOINT, 256), jnp.float32),
                   jax.ShapeDtypeStruct((2, 256), jnp.float32)),
        scratch_shapes=[pltpu.VMEM((NPOINT, 256), jnp.float32)],
    )(gt, w3, lb)

    gl = params['bnl_g'][None, :]
    bl = params['bnl_b'][None, :]
    out = pl.pallas_call(
        _k5_body,
        grid=(8,),
        in_specs=[pl.BlockSpec((NPOINT, 256), lambda b: (b, 0)),
                  pl.BlockSpec((2, 256), lambda b: (0, 0)),
                  full_spec(gl), full_spec(bl)],
        out_specs=pl.BlockSpec((NPOINT, 256), lambda b: (b, 0)),
        out_shape=jax.ShapeDtypeStruct((8 * NPOINT, 256), jnp.float32),
    )(lin, stl, gl, bl)
    return out.reshape(8, NPOINT, 256)


def kernel(xyz, points, params):
    B = xyz.shape[0]
    xyz_p = xyz.transpose(0, 2, 1)
    pts_p = points.transpose(0, 2, 1)
    fps_idx, new_xyz = _fps_pallas(xyz_p)
    idx = _knn_pallas(xyz_p, new_xyz)
    N = xyz_p.shape[1]
    table = jnp.concatenate(
        [xyz_p, pts_p, jnp.zeros((B, N, 13), jnp.float32)], axis=2).reshape(B * N, 144)
    gidx = (idx + (jnp.arange(B, dtype=jnp.int32) * N)[:, None, None]).reshape(-1)
    G = _sc_gather_rows(table, gidx, 144)
    out = _mlp_pallas(G, new_xyz, params)
    return (new_xyz.transpose(0, 2, 1), out.transpose(0, 2, 1))


# speedup vs baseline: 1.0789x; 1.0004x over previous
import functools

import jax
import jax.numpy as jnp
from jax import lax
from jax.experimental import pallas as pl
from jax.experimental.pallas import tpu as pltpu
from jax.experimental.pallas import tpu_sc as plsc

EPS = 1e-5
NPOINT = 512
NSAMPLE = 32


def _fps_kernel(x_ref, y_ref, z_ref, far0_ref, idx_ref, cx_ref, cy_ref, cz_ref):
    B, N = x_ref.shape
    iota_l = jax.lax.broadcasted_iota(jnp.int32, (B, N), 1)
    iota_p = jax.lax.broadcasted_iota(jnp.int32, (B, NPOINT), 1)
    x = x_ref[...]
    y = y_ref[...]
    z = z_ref[...]

    def body(i, st):
        dist, far, idx_acc, cx_acc, cy_acc, cz_acc = st
        mask = iota_l == far
        cx = jnp.sum(jnp.where(mask, x, 0.0), axis=1, keepdims=True)
        cy = jnp.sum(jnp.where(mask, y, 0.0), axis=1, keepdims=True)
        cz = jnp.sum(jnp.where(mask, z, 0.0), axis=1, keepdims=True)
        sel = iota_p == i
        idx_acc = jnp.where(sel, far, idx_acc)
        cx_acc = jnp.where(sel, cx, cx_acc)
        cy_acc = jnp.where(sel, cy, cy_acc)
        cz_acc = jnp.where(sel, cz, cz_acc)
        dx = x - cx
        dy = y - cy
        dz = z - cz
        d = (dx * dx + dy * dy) + dz * dz
        dist = jnp.minimum(dist, d)
        m = jnp.max(dist, axis=1, keepdims=True)
        far = jnp.min(jnp.where(dist == m, iota_l, N), axis=1, keepdims=True)
        return (dist, far, idx_acc, cx_acc, cy_acc, cz_acc)

    dist0 = jnp.full((B, N), 1e10, dtype=jnp.float32)
    zp = jnp.zeros((B, NPOINT), dtype=jnp.float32)
    zi = jnp.zeros((B, NPOINT), dtype=jnp.int32)
    _, _, idx_acc, cx_acc, cy_acc, cz_acc = jax.lax.fori_loop(
        0, NPOINT, body, (dist0, far0_ref[...], zi, zp, zp, zp))
    idx_ref[...] = idx_acc
    cx_ref[...] = cx_acc
    cy_ref[...] = cy_acc
    cz_ref[...] = cz_acc


def _fps_pallas(xyz_p):
    B, N, _ = xyz_p.shape
    far0 = jax.random.randint(jax.random.key(42), (B,), 0, N).astype(jnp.int32)[:, None]
    x = xyz_p[:, :, 0]
    y = xyz_p[:, :, 1]
    z = xyz_p[:, :, 2]
    idx, cx, cy, cz = pl.pallas_call(
        _fps_kernel,
        out_shape=(
            jax.ShapeDtypeStruct((B, NPOINT), jnp.int32),
            jax.ShapeDtypeStruct((B, NPOINT), jnp.float32),
            jax.ShapeDtypeStruct((B, NPOINT), jnp.float32),
            jax.ShapeDtypeStruct((B, NPOINT), jnp.float32),
        ),
    )(x, y, z, far0)
    new_xyz = jnp.stack([cx, cy, cz], axis=2)
    return idx, new_xyz


def _knn_kernel(nx8_ref, xyzT8_ref, idx_ref):
    M, N = 512, 4096
    nx8 = nx8_ref[0]
    xyzT8 = xyzT8_ref[0]
    mm = jax.lax.dot_general(nx8, xyzT8, (((1,), (0,)), ((), ())),
                             preferred_element_type=jnp.float32)
    sqr = -2.0 * mm
    sqr = sqr + jnp.sum(nx8 * nx8, axis=1, keepdims=True)
    sqr = sqr + jnp.sum(xyzT8 * xyzT8, axis=0, keepdims=True)
    CH, L = 32, 128
    INF = jnp.float32(jnp.inf)
    iota_l = jax.lax.broadcasted_iota(jnp.int32, (M, N), 1)
    lane_i = jax.lax.broadcasted_iota(jnp.int32, (M, L), 1)
    mins = [jnp.min(sqr[:, c * L:(c + 1) * L], axis=1, keepdims=True) for c in range(CH)]
    Mm = jnp.concatenate(mins, axis=1)
    ch_iota = jax.lax.broadcasted_iota(jnp.int32, (M, CH), 1)
    cols = []
    for _ in range(NSAMPLE):
        cM = jnp.argmin(Mm, axis=1).astype(jnp.int32)[:, None]
        Y = jnp.zeros((M, L), jnp.float32)
        for c in range(CH):
            Y = Y + jnp.where(cM == c, sqr[:, c * L:(c + 1) * L], 0.0)
        for e in cols:
            Y = jnp.where(lane_i == e - cM * L, INF, Y)
        lstar = jnp.argmin(Y, axis=1).astype(jnp.int32)[:, None]
        sel = cM * L + lstar
        cols.append(sel)
        newmin = jnp.min(jnp.where(lane_i == lstar, INF, Y), axis=1, keepdims=True)
        Mm = jnp.where(ch_iota == cM, newmin, Mm)
    idx_ref[0] = jnp.concatenate(cols, axis=1)


def _knn_pallas(xyz_p, new_xyz):
    B, N, _ = xyz_p.shape
    nx8 = jnp.concatenate([new_xyz, jnp.zeros((B, NPOINT, 5), jnp.float32)], axis=2)
    xyzT8 = jnp.concatenate([xyz_p.transpose(0, 2, 1), jnp.zeros((B, 5, N), jnp.float32)], axis=1)
    idx = pl.pallas_call(
        _knn_kernel,
        grid=(B,),
        in_specs=[
            pl.BlockSpec((1, NPOINT, 8), lambda b: (b, 0, 0)),
            pl.BlockSpec((1, 8, N), lambda b: (b, 0, 0)),
        ],
        out_specs=pl.BlockSpec((1, NPOINT, NSAMPLE), lambda b: (b, 0, 0)),
        out_shape=jax.ShapeDtypeStruct((B, NPOINT, NSAMPLE), jnp.int32),
    )(nx8, xyzT8)
    return idx


def _sc_gather_rows(table, gidx, ncols):
    NR = gidx.shape[0]
    NW = 32
    rows_per_w = NR // NW
    CHUNK = 128
    n_chunks = rows_per_w // CHUNK
    mesh = plsc.VectorSubcoreMesh(core_axis_name="c", subcore_axis_name="s")

    @functools.partial(
        pl.kernel,
        mesh=mesh,
        out_type=jax.ShapeDtypeStruct((NR, ncols), jnp.float32),
        scratch_types=[
            pltpu.VMEM((CHUNK,), jnp.int32),
            pltpu.VMEM((CHUNK, ncols), jnp.float32),
            pltpu.SemaphoreType.DMA,
        ],
        compiler_params=pltpu.CompilerParams(use_tc_tiling_on_sc=False),
    )
    def k(table_hbm, gidx_hbm, out_hbm, idx_v, rows_v, sem):
        wid = lax.axis_index("s") * 2 + lax.axis_index("c")
        base = wid * rows_per_w

        def chunk_body(ci, _):
            cb = base + ci * CHUNK
            pltpu.sync_copy(gidx_hbm.at[pl.ds(cb, CHUNK)], idx_v)
            pltpu.async_copy(table_hbm.at[idx_v], rows_v, sem).wait()
            pltpu.sync_copy(rows_v, out_hbm.at[pl.ds(cb, CHUNK)])
            return 0

        lax.fori_loop(0, n_chunks, chunk_body, 0)

    return k(table, gidx)


NTOT = float(8 * NPOINT * NSAMPLE)
TILE = 512


def _tile_stats(x):
    s = jnp.sum(x, axis=0, keepdims=True)
    sq = jnp.sum(x * x, axis=0, keepdims=True)
    return jnp.concatenate([s, sq], axis=0)


def _acc_stats(ref, x):
    @pl.when(pl.program_id(0) == 0)
    def _():
        ref[...] = jnp.zeros_like(ref)
    ref[...] += _tile_stats(x)


def _bn_consts(stats, g, b):
    mean = stats[0:1] / NTOT
    var = stats[1:2] / NTOT - mean * mean
    a = g / jnp.sqrt(var + EPS)
    c = b - mean * a
    return a, c


def _k0_body(g_ref, nxe_ref, wx_ref, wf_ref, b0_ref, wwx_ref, bw0_ref,
             out_ref, wout_ref, st_ref, wst_ref):
    gt = g_ref[...]
    xn16 = gt[:, 0:16] - nxe_ref[...]
    out0 = (jax.lax.dot_general(xn16, wx_ref[...], (((1,), (0,)), ((), ())),
                                preferred_element_type=jnp.float32)
            + jax.lax.dot_general(gt, wf_ref[...], (((1,), (0,)), ((), ())),
                                  preferred_element_type=jnp.float32)
            + b0_ref[...])
    wout0 = jax.lax.dot_general(xn16, wwx_ref[...], (((1,), (0,)), ((), ())),
                                preferred_element_type=jnp.float32) + bw0_ref[...]
    out_ref[...] = out0
    wout_ref[...] = wout0
    _acc_stats(st_ref, out0)
    _acc_stats(wst_ref, wout0)


def _klayer_body(x_ref, wx_in_ref, st_in_ref, wst_in_ref, g_ref, bb_ref,
                 wg_ref, wbb_ref, w_ref, b_ref, ww_ref, wb_ref,
                 out_ref, wout_ref, st_ref, wst_ref):
    a, c = _bn_consts(st_in_ref[...], g_ref[...], bb_ref[...])
    x = jnp.maximum(x_ref[...] * a + c, 0.0)
    wa, wc = _bn_consts(wst_in_ref[...], wg_ref[...], wbb_ref[...])
    wx = jnp.maximum(wx_in_ref[...] * wa + wc, 0.0)
    out = jax.lax.dot_general(x, w_ref[...], (((1,), (0,)), ((), ())),
                              preferred_element_type=jnp.float32) + b_ref[...]
    wout = jax.lax.dot_general(wx, ww_ref[...], (((1,), (0,)), ((), ())),
                               preferred_element_type=jnp.float32) + wb_ref[...]
    out_ref[...] = out
    wout_ref[...] = wout
    _acc_stats(st_ref, out)
    _acc_stats(wst_ref, wout)


_NB = 64


def _k3_body(x_ref, wx_ref, st_ref, wst_ref, g_ref, bb_ref, wg_ref, wbb_ref, gt_ref):
    a, c = _bn_consts(st_ref[...], g_ref[...], bb_ref[...])
    x = jnp.maximum(x_ref[...] * a + c, 0.0)
    wa, wc = _bn_consts(wst_ref[...], wg_ref[...], wbb_ref[...])
    wx = jnp.maximum(wx_ref[...] * wa + wc, 0.0)
    for i in range(_NB):
        xi = x[i * NSAMPLE:(i + 1) * NSAMPLE]
        wi = wx[i * NSAMPLE:(i + 1) * NSAMPLE]
        gt_ref[:, i, :] = jax.lax.dot_general(
            wi, xi, (((0,), (0,)), ((), ())), preferred_element_type=jnp.float32)


def _k4_body(gt_ref, w3_ref, lb_ref, lin_ref, st_ref, acc_ref):
    b = pl.program_id(0)
    j = pl.program_id(1)

    @pl.when(j == 0)
    def _():
        acc_ref[...] = jnp.broadcast_to(lb_ref[...], acc_ref.shape)

    acc_ref[...] += jax.lax.dot_general(
        gt_ref[0], w3_ref[0], (((1,), (0,)), ((), ())),
        preferred_element_type=jnp.float32)

    @pl.when(j == 15)
    def _():
        a = acc_ref[...]
        lin_ref[...] = a

        @pl.when(b == 0)
        def _():
            st_ref[...] = jnp.zeros_like(st_ref)

        st_ref[...] += _tile_stats(a)


def _k5_body(x_ref, st_ref, g_ref, bb_ref, o_ref):
    a, c = _bn_consts(st_ref[...] * (NTOT / (8.0 * NPOINT)), g_ref[...], bb_ref[...])
    o_ref[...] = jnp.maximum(x_ref[...] * a + c, 0.0)


def _mlp_pallas(G, new_xyz, params):
    R = G.shape[0]
    nsteps = R // TILE

    def pad_rows(m, rows, at, total):
        z0 = jnp.zeros((at, m.shape[1]), jnp.float32)
        z1 = jnp.zeros((total - at - rows, m.shape[1]), jnp.float32)
        return jnp.concatenate([z0, m, z1], axis=0)

    def pad_cols(v, total):
        return jnp.concatenate([v, jnp.zeros((total - v.shape[0],), jnp.float32)])

    w0t = params['conv0_w'].T
    wx16 = pad_rows(w0t[0:3], 3, 0, 16)
    wf144 = pad_rows(w0t[3:131], 128, 3, 144)
    b0 = params['conv0_b'][None, :]
    ww0 = jnp.pad(params['wconv0_w'].T, ((0, 13), (0, 8)))
    bw0 = pad_cols(params['wconv0_b'], 16)[None, :]
    nxe = jnp.repeat(
        jnp.concatenate([new_xyz, jnp.zeros((8, NPOINT, 13), jnp.float32)],
                        axis=2).reshape(8 * NPOINT, 16), NSAMPLE, axis=0)

    row_spec = lambda w: pl.BlockSpec((TILE, w), lambda i: (i, 0))
    full_spec = lambda a: pl.BlockSpec(a.shape, lambda i: tuple(0 for _ in a.shape))
    stat_spec = lambda w: pl.BlockSpec((2, w), lambda i: (0, 0))

    out0, wout0, st0, wst0 = pl.pallas_call(
        _k0_body,
        grid=(nsteps,),
        in_specs=[row_spec(144), row_spec(16)] + [full_spec(a) for a in (wx16, wf144, b0, ww0, bw0)],
        out_specs=(row_spec(128), row_spec(16), stat_spec(128), stat_spec(16)),
        out_shape=(jax.ShapeDtypeStruct((R, 128), jnp.float32),
                   jax.ShapeDtypeStruct((R, 16), jnp.float32),
                   jax.ShapeDtypeStruct((2, 128), jnp.float32),
                   jax.ShapeDtypeStruct((2, 16), jnp.float32)),
    )(G, nxe, wx16, wf144, b0, ww0, bw0)

    def layer(i, x, wx, st, wst, oc):
        ic = x.shape[1]
        g = params['bn%d_g' % (i - 1)][None, :]
        bb = params['bn%d_b' % (i - 1)][None, :]
        wg = pad_cols(params['wbn%d_g' % (i - 1)], 16)[None, :]
        wbb = pad_cols(params['wbn%d_b' % (i - 1)], 16)[None, :]
        w = params['conv%d_w' % i].T
        b = params['conv%d_b' % i][None, :]
        wwt = params['wconv%d_w' % i].T
        ww = jnp.pad(wwt, ((0, 16 - wwt.shape[0]), (0, 16 - wwt.shape[1])))
        wb = pad_cols(params['wconv%d_b' % i], 16)[None, :]
        return pl.pallas_call(
            _klayer_body,
            grid=(nsteps,),
            in_specs=[row_spec(ic), row_spec(16), stat_spec(ic), stat_spec(16)]
                     + [full_spec(a) for a in (g, bb, wg, wbb, w, b, ww, wb)],
            out_specs=(row_spec(oc), row_spec(16), stat_spec(oc), stat_spec(16)),
            out_shape=(jax.ShapeDtypeStruct((R, oc), jnp.float32),
                       jax.ShapeDtypeStruct((R, 16), jnp.float32),
                       jax.ShapeDtypeStruct((2, oc), jnp.float32),
                       jax.ShapeDtypeStruct((2, 16), jnp.float32)),
        )(x, wx, st, wst, g, bb, wg, wbb, w, b, ww, wb)

    out1, wout1, st1, wst1 = layer(1, out0, wout0, st0, wst0, 128)
    out2, wout2, st2, wst2 = layer(2, out1, wout1, st1, wst1, 256)

    g2 = params['bn2_g'][None, :]
    bb2 = params['bn2_b'][None, :]
    wg2 = pad_cols(params['wbn2_g'], 16)[None, :]
    wbb2 = pad_cols(params['wbn2_b'], 16)[None, :]
    n_total = 8 * NPOINT
    gt = pl.pallas_call(
        _k3_body,
        grid=(n_total // _NB,),
        in_specs=[pl.BlockSpec((_NB * NSAMPLE, 256), lambda i: (i, 0)),
                  pl.BlockSpec((_NB * NSAMPLE, 16), lambda i: (i, 0)),
                  stat_spec(256), stat_spec(16),
                  full_spec(g2), full_spec(bb2), full_spec(wg2), full_spec(wbb2)],
        out_specs=pl.BlockSpec((16, _NB, 256), lambda i: (0, i, 0)),
        out_shape=jax.ShapeDtypeStruct((16, n_total, 256), jnp.float32),
    )(out2, wout2, st2, wst2, g2, bb2, wg2, wbb2)

    w3 = params['lin_w'].reshape(256, 256, 16).transpose(2, 1, 0)
    lb = params['lin_b'][None, :]
    lin, stl = pl.pallas_call(
        _k4_body,
        grid=(8, 16),
        in_specs=[pl.BlockSpec((1, NPOINT, 256), lambda b, j: (j, b, 0)),
                  pl.BlockSpec((1, 256, 256), lambda b, j: (j, 0, 0)),
                  pl.BlockSpec((1, 256), lambda b, j: (0, 0))],
        out_specs=(pl.BlockSpec((NPOINT, 256), lambda b, j: (b, 0)),
                   pl.BlockSpec((2, 256), lambda b, j: (0, 0))),
        out_shape=(jax.ShapeDtypeStruct((8 * NPOINT, 256), jnp.float32),
                   jax.ShapeDtypeStruct((2, 256), jnp.float32)),
        scratch_shapes=[pltpu.VMEM((NPOINT, 256), jnp.float32)],
    )(gt, w3, lb)

    gl = params['bnl_g'][None, :]
    bl = params['bnl_b'][None, :]
    out = pl.pallas_call(
        _k5_body,
        grid=(8,),
        in_specs=[pl.BlockSpec((NPOINT, 256), lambda b: (b, 0)),
                  pl.BlockSpec((2, 256), lambda b: (0, 0)),
                  full_spec(gl), full_spec(bl)],
        out_specs=pl.BlockSpec((NPOINT, 256), lambda b: (b, 0)),
        out_shape=jax.ShapeDtypeStruct((8 * NPOINT, 256), jnp.float32),
    )(lin, stl, gl, bl)
    return out.reshape(8, NPOINT, 256)


def kernel(xyz, points, params):
    B = xyz.shape[0]
    xyz_p = xyz.transpose(0, 2, 1)
    pts_p = points.transpose(0, 2, 1)
    _, new_xyz = _fps_pallas(xyz_p)
    idx = _knn_pallas(xyz_p, new_xyz)
    N = xyz_p.shape[1]
    table = jnp.concatenate(
        [xyz_p, pts_p, jnp.zeros((B, N, 13), jnp.float32)], axis=2).reshape(B * N, 144)
    gidx = (idx + (jnp.arange(B, dtype=jnp.int32) * N)[:, None, None]).reshape(-1)
    G = _sc_gather_rows(table, gidx, 144)
    out = _mlp_pallas(G, new_xyz, params)
    return (new_xyz.transpose(0, 2, 1), out.transpose(0, 2, 1))
